# Initial kernel scaffold; baseline (speedup 1.0000x reference)
import jax
import jax.numpy as jnp
from jax.experimental import pallas as pl


def kernel(x, nodes, nbs_idx, self_weight, nb_weight, bias):
    raise NotImplementedError("write your pallas kernel here")


# speedup vs baseline: 1.2641x; 1.2641x over previous
import functools

import jax
import jax.numpy as jnp
from jax import lax
from jax.experimental import pallas as pl
from jax.experimental.pallas import tpu as pltpu
from jax.experimental.pallas import tpu_sc as plsc

N_NODES_C = 50000
D = 128
K_NBS = 32
NW = 32
CHUNK = 8
VPR = D // 16


def _sc_gather_pool(x, nodes_p, nbs_flat, bp):
    b_per_w = bp // NW
    n_chunks = b_per_w // CHUNK
    mesh = plsc.VectorSubcoreMesh(core_axis_name="c", subcore_axis_name="s")

    @functools.partial(
        pl.kernel,
        mesh=mesh,
        out_type=[
            jax.ShapeDtypeStruct((bp, D), jnp.float32),
            jax.ShapeDtypeStruct((bp, D), jnp.float32),
        ],
        scratch_types=[
            pltpu.VMEM((b_per_w,), jnp.int32),
            pltpu.VMEM((b_per_w * K_NBS,), jnp.int32),
            pltpu.VMEM((CHUNK * K_NBS, D), jnp.float32),
            pltpu.VMEM((CHUNK, D), jnp.float32),
            pltpu.VMEM((CHUNK, D), jnp.float32),
            pltpu.SemaphoreType.DMA,
            pltpu.SemaphoreType.DMA,
        ],
    )
    def sc_kernel(x_hbm, nodes_hbm, nbs_hbm, self_hbm, pooled_hbm,
                  nid_v, nbid_v, rows_v, self_v, pooled_v, sem_n, sem_s):
        wid = lax.axis_index("s") * 2 + lax.axis_index("c")
        wbase = wid * b_per_w
        pltpu.sync_copy(nodes_hbm.at[pl.ds(wbase, b_per_w)], nid_v)
        pltpu.sync_copy(nbs_hbm.at[pl.ds(wbase * K_NBS, b_per_w * K_NBS)],
                        nbid_v)

        def chunk_body(c, carry):
            base = c * CHUNK
            cp_n = pltpu.async_copy(
                x_hbm.at[nbid_v.at[pl.ds(base * K_NBS, CHUNK * K_NBS)]],
                rows_v, sem_n)
            cp_s = pltpu.async_copy(
                x_hbm.at[nid_v.at[pl.ds(base, CHUNK)]], self_v, sem_s)
            cp_n.wait()
            cp_s.wait()
            for t in range(CHUNK):
                r0 = t * K_NBS
                accs = tuple(rows_v[r0, pl.ds(v * 16, 16)] for v in range(VPR))

                def nb_body(j, a):
                    return tuple(a[v] + rows_v[r0 + j, pl.ds(v * 16, 16)]
                                 for v in range(VPR))

                accs = lax.fori_loop(1, K_NBS, nb_body, accs)
                for v in range(VPR):
                    pooled_v[t, pl.ds(v * 16, 16)] = accs[v]
            gbase = wbase + base
            pltpu.sync_copy(self_v, self_hbm.at[pl.ds(gbase, CHUNK)])
            pltpu.sync_copy(pooled_v, pooled_hbm.at[pl.ds(gbase, CHUNK)])
            return carry

        lax.fori_loop(0, n_chunks, chunk_body, 0)

    return sc_kernel(x, nodes_p, nbs_flat)


def _tc_combine_body(self_ref, pooled_ref, ws_ref, wn_ref, b_ref, o_ref):
    acc = jnp.dot(self_ref[...], ws_ref[...], preferred_element_type=jnp.float32)
    acc = acc + jnp.dot(pooled_ref[...], wn_ref[...],
                        preferred_element_type=jnp.float32)
    o_ref[...] = jnp.maximum(acc + b_ref[...], 0.0)


def _tc_combine(self_rows, pooled, ws, wn_scaled, bias2d, bp):
    bk = 1280
    grid = (bp // bk,)
    return pl.pallas_call(
        _tc_combine_body,
        grid=grid,
        in_specs=[
            pl.BlockSpec((bk, D), lambda i: (i, 0)),
            pl.BlockSpec((bk, D), lambda i: (i, 0)),
            pl.BlockSpec((D, D), lambda i: (0, 0)),
            pl.BlockSpec((D, D), lambda i: (0, 0)),
            pl.BlockSpec((1, D), lambda i: (0, 0)),
        ],
        out_specs=pl.BlockSpec((bk, D), lambda i: (i, 0)),
        out_shape=jax.ShapeDtypeStruct((bp, D), jnp.float32),
    )(self_rows, pooled, ws, wn_scaled, bias2d)


def kernel(x, nodes, nbs_idx, self_weight, nb_weight, bias):
    b = nodes.shape[0]
    bp = ((b + 8 * NW - 1) // (8 * NW)) * (8 * NW)
    nodes_p = jnp.pad(nodes.astype(jnp.int32), (0, bp - b))
    nbs_flat = jnp.pad(nbs_idx.astype(jnp.int32), ((0, bp - b), (0, 0))).reshape(-1)
    self_rows, pooled = _sc_gather_pool(x, nodes_p, nbs_flat, bp)
    out = _tc_combine(self_rows, pooled, self_weight,
                      nb_weight * (1.0 / K_NBS), bias.reshape(1, D), bp)
    return out[:b]


# speedup vs baseline: 1.4097x; 1.1152x over previous
import functools

import jax
import jax.numpy as jnp
from jax import lax
from jax.experimental import pallas as pl
from jax.experimental.pallas import tpu as pltpu
from jax.experimental.pallas import tpu_sc as plsc

D = 128
K_NBS = 32
NW = 32
CHUNK = 8
VPR = D // 16
ROWS = CHUNK * K_NBS


def _accumulate_chunk(rows_v, pooled_v):
    for t in range(CHUNK):
        r0 = t * K_NBS

        def nb_body(i, a):
            for u in range(4):
                r = r0 + i * 4 + u
                a = tuple(a[v] + rows_v[r, pl.ds(v * 16, 16)]
                          for v in range(VPR))
            return a

        zero = jnp.zeros((16,), jnp.float32)
        accs = lax.fori_loop(0, K_NBS // 4, nb_body, (zero,) * VPR)
        for v in range(VPR):
            pooled_v[t, pl.ds(v * 16, 16)] = accs[v]


def _sc_gather_pool(x, nodes_p, nbs_flat, bp):
    b_per_w = bp // NW
    n_chunks = b_per_w // CHUNK
    n_pairs = n_chunks // 2
    mesh = plsc.VectorSubcoreMesh(core_axis_name="c", subcore_axis_name="s")

    @functools.partial(
        pl.kernel,
        mesh=mesh,
        out_type=[
            jax.ShapeDtypeStruct((bp, D), jnp.float32),
            jax.ShapeDtypeStruct((bp, D), jnp.float32),
        ],
        scratch_types=[
            pltpu.VMEM((b_per_w,), jnp.int32),
            pltpu.VMEM((b_per_w * K_NBS,), jnp.int32),
            pltpu.VMEM((ROWS, D), jnp.float32),
            pltpu.VMEM((ROWS, D), jnp.float32),
            pltpu.VMEM((CHUNK, D), jnp.float32),
            pltpu.VMEM((CHUNK, D), jnp.float32),
            pltpu.VMEM((b_per_w, D), jnp.float32),
            pltpu.SemaphoreType.DMA,
            pltpu.SemaphoreType.DMA,
            pltpu.SemaphoreType.DMA,
            pltpu.SemaphoreType.DMA,
            pltpu.SemaphoreType.DMA,
        ],
    )
    def sc_kernel(x_hbm, nodes_hbm, nbs_hbm, self_hbm, pooled_hbm,
                  nid_v, nbid_v, rows0, rows1, pooled0, pooled1, selfr_v,
                  sem_g0, sem_g1, sem_p0, sem_p1, sem_s):
        wid = lax.axis_index("s") * 2 + lax.axis_index("c")
        wbase = wid * b_per_w
        pltpu.sync_copy(nodes_hbm.at[pl.ds(wbase, b_per_w)], nid_v)
        pltpu.sync_copy(nbs_hbm.at[pl.ds(wbase * K_NBS, b_per_w * K_NBS)],
                        nbid_v)
        cp_self = pltpu.async_copy(x_hbm.at[nid_v], selfr_v, sem_s)

        rows = (rows0, rows1)
        pooled = (pooled0, pooled1)
        sem_g = (sem_g0, sem_g1)
        sem_p = (sem_p0, sem_p1)

        def start_gather(c, par):
            pltpu.async_copy(
                x_hbm.at[nbid_v.at[pl.ds(c * ROWS, ROWS)]],
                rows[par], sem_g[par])

        def wait_gather(par):
            pltpu.make_async_copy(x_hbm.at[nbid_v.at[pl.ds(0, ROWS)]],
                                  rows[par], sem_g[par]).wait()

        def start_pooled_out(c, par):
            pltpu.async_copy(
                pooled[par], pooled_hbm.at[pl.ds(wbase + c * CHUNK, CHUNK)],
                sem_p[par])

        def wait_pooled_out(par):
            pltpu.make_async_copy(
                pooled[par], pooled_hbm.at[pl.ds(wbase, CHUNK)],
                sem_p[par]).wait()

        start_gather(0, 0)

        def pair_body(p, carry):
            c0 = p * 2

            @pl.when(c0 + 1 < n_chunks)
            def _():
                start_gather(c0 + 1, 1)

            wait_gather(0)

            @pl.when(p > 0)
            def _():
                wait_pooled_out(0)

            _accumulate_chunk(rows0, pooled0)
            start_pooled_out(c0, 0)

            @pl.when(c0 + 2 < n_chunks)
            def _():
                start_gather(c0 + 2, 0)

            wait_gather(1)

            @pl.when(p > 0)
            def _():
                wait_pooled_out(1)

            _accumulate_chunk(rows1, pooled1)
            start_pooled_out(c0 + 1, 1)
            return carry

        lax.fori_loop(0, n_pairs, pair_body, 0)
        wait_pooled_out(0)
        wait_pooled_out(1)
        cp_self.wait()
        pltpu.sync_copy(selfr_v, self_hbm.at[pl.ds(wbase, b_per_w)])

    return sc_kernel(x, nodes_p, nbs_flat)


def _tc_combine_body(self_ref, pooled_ref, ws_ref, wn_ref, b_ref, o_ref):
    acc = jnp.dot(self_ref[...], ws_ref[...], preferred_element_type=jnp.float32)
    acc = acc + jnp.dot(pooled_ref[...], wn_ref[...],
                        preferred_element_type=jnp.float32)
    o_ref[...] = jnp.maximum(acc + b_ref[...], 0.0)


def _tc_combine(self_rows, pooled, ws, wn_scaled, bias2d, bp):
    bk = 1280
    grid = (bp // bk,)
    return pl.pallas_call(
        _tc_combine_body,
        grid=grid,
        in_specs=[
            pl.BlockSpec((bk, D), lambda i: (i, 0)),
            pl.BlockSpec((bk, D), lambda i: (i, 0)),
            pl.BlockSpec((D, D), lambda i: (0, 0)),
            pl.BlockSpec((D, D), lambda i: (0, 0)),
            pl.BlockSpec((1, D), lambda i: (0, 0)),
        ],
        out_specs=pl.BlockSpec((bk, D), lambda i: (i, 0)),
        out_shape=jax.ShapeDtypeStruct((bp, D), jnp.float32),
    )(self_rows, pooled, ws, wn_scaled, bias2d)


def kernel(x, nodes, nbs_idx, self_weight, nb_weight, bias):
    b = nodes.shape[0]
    bp = ((b + 8 * NW - 1) // (8 * NW)) * (8 * NW)
    nodes_p = jnp.pad(nodes.astype(jnp.int32), (0, bp - b))
    nbs_flat = jnp.pad(nbs_idx.astype(jnp.int32), ((0, bp - b), (0, 0))).reshape(-1)
    self_rows, pooled = _sc_gather_pool(x, nodes_p, nbs_flat, bp)
    out = _tc_combine(self_rows, pooled, self_weight,
                      nb_weight * (1.0 / K_NBS), bias.reshape(1, D), bp)
    return out[:b]


# speedup vs baseline: 1.4368x; 1.0192x over previous
import functools

import jax
import jax.numpy as jnp
from jax import lax
from jax.experimental import pallas as pl
from jax.experimental.pallas import tpu as pltpu
from jax.experimental.pallas import tpu_sc as plsc

D = 128
K_NBS = 32
NS = 16
CHUNK = 8
VPR = D // 16
ROWS = CHUNK * K_NBS
BP = 10240
Q0 = 528
Q1 = BP // NS - Q0


def _accumulate_chunk(rows_v, pooled_v):
    for t in range(CHUNK):
        r0 = t * K_NBS

        def nb_body(i, a):
            for u in range(4):
                r = r0 + i * 4 + u
                a = tuple(a[v] + rows_v[r, pl.ds(v * 16, 16)]
                          for v in range(VPR))
            return a

        zero = jnp.zeros((16,), jnp.float32)
        accs = lax.fori_loop(0, K_NBS // 4, nb_body, (zero,) * VPR)
        for v in range(VPR):
            pooled_v[t, pl.ds(v * 16, 16)] = accs[v]


def _sc_gather_pool(x, nodes_p, nbs_flat):
    qmax = max(Q0, Q1)
    mesh = plsc.VectorSubcoreMesh(core_axis_name="c", subcore_axis_name="s")

    @functools.partial(
        pl.kernel,
        mesh=mesh,
        out_type=[
            jax.ShapeDtypeStruct((BP, D), jnp.float32),
            jax.ShapeDtypeStruct((BP, D), jnp.float32),
        ],
        scratch_types=[
            pltpu.VMEM((qmax,), jnp.int32),
            pltpu.VMEM((qmax * K_NBS,), jnp.int32),
            pltpu.VMEM((ROWS, D), jnp.float32),
            pltpu.VMEM((ROWS, D), jnp.float32),
            pltpu.VMEM((CHUNK, D), jnp.float32),
            pltpu.VMEM((CHUNK, D), jnp.float32),
            pltpu.VMEM((CHUNK, D), jnp.float32),
            pltpu.VMEM((CHUNK, D), jnp.float32),
            pltpu.SemaphoreType.DMA,
            pltpu.SemaphoreType.DMA,
            pltpu.SemaphoreType.DMA,
            pltpu.SemaphoreType.DMA,
            pltpu.SemaphoreType.DMA,
            pltpu.SemaphoreType.DMA,
            pltpu.SemaphoreType.DMA,
            pltpu.SemaphoreType.DMA,
        ],
    )
    def sc_kernel(x_hbm, nodes_hbm, nbs_hbm, self_hbm, pooled_hbm,
                  nid_v, nbid_v, rows0, rows1, pooled0, pooled1, selfb0, selfb1,
                  sem_g0, sem_g1, sem_p0, sem_p1,
                  sem_sg0, sem_sg1, sem_sw0, sem_sw1):
        cid = lax.axis_index("c")
        sid = lax.axis_index("s")
        rows = (rows0, rows1)
        pooled = (pooled0, pooled1)
        selfb = (selfb0, selfb1)
        sem_g = (sem_g0, sem_g1)
        sem_p = (sem_p0, sem_p1)
        sem_sg = (sem_sg0, sem_sg1)
        sem_sw = (sem_sw0, sem_sw1)

        def worker(q, wbase):
            n_chunks = q // CHUNK
            n_pairs = n_chunks // 2
            pltpu.sync_copy(nodes_hbm.at[pl.ds(wbase, q)],
                            nid_v.at[pl.ds(0, q)])
            pltpu.sync_copy(nbs_hbm.at[pl.ds(wbase * K_NBS, q * K_NBS)],
                            nbid_v.at[pl.ds(0, q * K_NBS)])

            def start_gathers(c, par):
                pltpu.async_copy(
                    x_hbm.at[nbid_v.at[pl.ds(c * ROWS, ROWS)]],
                    rows[par], sem_g[par])
                pltpu.async_copy(
                    x_hbm.at[nid_v.at[pl.ds(c * CHUNK, CHUNK)]],
                    selfb[par], sem_sg[par])

            def wait_gather(par):
                pltpu.make_async_copy(
                    x_hbm.at[nbid_v.at[pl.ds(0, ROWS)]],
                    rows[par], sem_g[par]).wait()

            def finish_chunk(c, par, first):
                wait_gather(par)

                @pl.when(jnp.logical_not(first))
                def _():
                    pltpu.make_async_copy(
                        pooled[par], pooled_hbm.at[pl.ds(wbase, CHUNK)],
                        sem_p[par]).wait()

                _accumulate_chunk(rows[par], pooled[par])
                pltpu.async_copy(
                    pooled[par],
                    pooled_hbm.at[pl.ds(wbase + c * CHUNK, CHUNK)],
                    sem_p[par])
                pltpu.make_async_copy(
                    x_hbm.at[nid_v.at[pl.ds(0, CHUNK)]],
                    selfb[par], sem_sg[par]).wait()

                @pl.when(jnp.logical_not(first))
                def _():
                    pltpu.make_async_copy(
                        selfb[par], self_hbm.at[pl.ds(wbase, CHUNK)],
                        sem_sw[par]).wait()

                pltpu.async_copy(
                    selfb[par],
                    self_hbm.at[pl.ds(wbase + c * CHUNK, CHUNK)],
                    sem_sw[par])

            start_gathers(0, 0)

            def pair_body(p, carry):
                c0 = p * 2

                @pl.when(c0 + 1 < n_chunks)
                def _():
                    start_gathers(c0 + 1, 1)

                finish_chunk(c0, 0, p == 0)

                @pl.when(c0 + 2 < n_chunks)
                def _():
                    start_gathers(c0 + 2, 0)

                finish_chunk(c0 + 1, 1, p == 0)
                return carry

            lax.fori_loop(0, n_pairs, pair_body, 0)
            for par in range(2):
                pltpu.make_async_copy(
                    pooled[par], pooled_hbm.at[pl.ds(wbase, CHUNK)],
                    sem_p[par]).wait()
                pltpu.make_async_copy(
                    selfb[par], self_hbm.at[pl.ds(wbase, CHUNK)],
                    sem_sw[par]).wait()

        @pl.when(cid == 0)
        def _():
            worker(Q0, sid * Q0)

        @pl.when(cid == 1)
        def _():
            worker(Q1, NS * Q0 + sid * Q1)

    return sc_kernel(x, nodes_p, nbs_flat)


def _tc_combine_body(self_ref, pooled_ref, ws_ref, wn_ref, b_ref, o_ref):
    acc = jnp.dot(self_ref[...], ws_ref[...], preferred_element_type=jnp.float32)
    acc = acc + jnp.dot(pooled_ref[...], wn_ref[...],
                        preferred_element_type=jnp.float32)
    o_ref[...] = jnp.maximum(acc + b_ref[...], 0.0)


def _tc_combine(self_rows, pooled, ws, wn_scaled, bias2d):
    bk = 1280
    grid = (BP // bk,)
    return pl.pallas_call(
        _tc_combine_body,
        grid=grid,
        in_specs=[
            pl.BlockSpec((bk, D), lambda i: (i, 0)),
            pl.BlockSpec((bk, D), lambda i: (i, 0)),
            pl.BlockSpec((D, D), lambda i: (0, 0)),
            pl.BlockSpec((D, D), lambda i: (0, 0)),
            pl.BlockSpec((1, D), lambda i: (0, 0)),
        ],
        out_specs=pl.BlockSpec((bk, D), lambda i: (i, 0)),
        out_shape=jax.ShapeDtypeStruct((BP, D), jnp.float32),
    )(self_rows, pooled, ws, wn_scaled, bias2d)


def kernel(x, nodes, nbs_idx, self_weight, nb_weight, bias):
    b = nodes.shape[0]
    nodes_p = jnp.pad(nodes.astype(jnp.int32), (0, BP - b))
    nbs_flat = jnp.pad(nbs_idx.astype(jnp.int32), ((0, BP - b), (0, 0))).reshape(-1)
    self_rows, pooled = _sc_gather_pool(x, nodes_p, nbs_flat)
    out = _tc_combine(self_rows, pooled, self_weight,
                      nb_weight * (1.0 / K_NBS), bias.reshape(1, D))
    return out[:b]
